# baseline (speedup 1.0000x reference)
import jax
import jax.numpy as jnp
from jax import lax
from jax.experimental import pallas as pl
from jax.experimental.pallas import tpu as pltpu
from jax.experimental.pallas import tpu_sc as plsc

N_FACTORS = 16
NC = 2
NS = 16
NW = NC * NS
L = 16
CH = 16


def _mf_body(iu_hbm, ii_hbm, ut_hbm, vt_hbm, w_hbm, b_hbm, out_hbm,
             iu_v, ii_v, ring, ucol_v, w_v, b_v, out_v, sem_a, sem_b):
    b_per_w = iu_v.shape[0]
    nch = b_per_w // CH
    wid = lax.axis_index("s") * NC + lax.axis_index("c")
    base = pl.multiple_of(wid * b_per_w, b_per_w)

    pltpu.sync_copy(iu_hbm.at[pl.ds(base, b_per_w)], iu_v)
    pltpu.sync_copy(ii_hbm.at[pl.ds(base, b_per_w)], ii_v)
    pltpu.sync_copy(w_hbm, w_v)
    pltpu.sync_copy(b_hbm, b_v)

    w_bcast = [w_v[f, :] for f in range(N_FACTORS)]
    bvec = b_v[...]
    iota = lax.iota(jnp.int32, L)

    def issue(idx_v, t_hbm, c, half, sem):
        ivec = idx_v[pl.ds(c * CH, CH)]
        for j in range(CH):
            blk = pl.multiple_of(
                lax.shift_left(lax.shift_right_logical(ivec[j], 7), 7), 128)
            pltpu.make_async_copy(
                t_hbm.at[:, pl.ds(blk, 128)],
                ring.at[half * CH + j], sem).start()

    def drain(half, sem):
        for j in range(CH):
            pltpu.make_async_copy(
                ut_hbm.at[:, pl.ds(0, 128)],
                ring.at[half * CH + j], sem).wait()

    def compute_u(c, half):
        uvec = iu_v[pl.ds(c * CH, CH)]
        umod = lax.bitwise_and(uvec, jnp.int32(127))
        slots = half * CH + iota
        for f in range(N_FACTORS):
            fs = jnp.full((L,), f, jnp.int32)
            ucol_v[f, pl.ds(c * CH, CH)] = plsc.load_gather(
                ring, [slots, fs, umod])

    def compute_v(c, half):
        vvec = ii_v[pl.ds(c * CH, CH)]
        vmod = lax.bitwise_and(vvec, jnp.int32(127))
        slots = half * CH + iota
        acc = bvec
        for f in range(N_FACTORS):
            fs = jnp.full((L,), f, jnp.int32)
            gv = plsc.load_gather(ring, [slots, fs, vmod])
            gu = ucol_v[f, pl.ds(c * CH, CH)]
            acc = acc + gu * gv * w_bcast[f]
        out_v[pl.ds(c * CH, CH)] = acc

    for idx_v, t_hbm, compute in (
        (iu_v, ut_hbm, compute_u),
        (ii_v, vt_hbm, compute_v),
    ):
        issue(idx_v, t_hbm, 0, 0, sem_a)

        def pair_body(k, carry, idx_v=idx_v, t_hbm=t_hbm, compute=compute):
            issue(idx_v, t_hbm, 2 * k + 1, 1, sem_b)
            drain(0, sem_a)
            compute(2 * k, 0)
            issue(idx_v, t_hbm, jnp.minimum(2 * k + 2, nch - 1), 0, sem_a)
            drain(1, sem_b)
            compute(2 * k + 1, 1)
            return carry

        lax.fori_loop(0, nch // 2, pair_body, 0)
        drain(0, sem_a)

    pltpu.sync_copy(out_v, out_hbm.at[pl.ds(base, b_per_w)])


@jax.jit
def kernel(idx_users, idx_items, user_emb_mf, item_emb_mf, W_out, b_out):
    B = idx_users.shape[0]
    b_per_w = B // NW
    mesh = plsc.VectorSubcoreMesh(core_axis_name="c", subcore_axis_name="s",
                                  num_cores=NC, num_subcores=NS)
    k = pl.kernel(
        _mf_body,
        out_type=jax.ShapeDtypeStruct((B,), jnp.float32),
        mesh=mesh,
        scratch_types=[
            pltpu.VMEM((b_per_w,), jnp.int32),
            pltpu.VMEM((b_per_w,), jnp.int32),
            pltpu.VMEM((2 * CH, N_FACTORS, 128), jnp.float32),
            pltpu.VMEM((N_FACTORS, b_per_w), jnp.float32),
            pltpu.VMEM((N_FACTORS, L), jnp.float32),
            pltpu.VMEM((L,), jnp.float32),
            pltpu.VMEM((b_per_w,), jnp.float32),
            pltpu.SemaphoreType.DMA,
            pltpu.SemaphoreType.DMA,
        ],
        compiler_params=pltpu.CompilerParams(
            needs_layout_passes=False, use_tc_tiling_on_sc=True),
    )
    w16 = jnp.broadcast_to(
        W_out.reshape((N_FACTORS, 1)).astype(jnp.float32), (N_FACTORS, L))
    b16 = jnp.broadcast_to(b_out.reshape(()).astype(jnp.float32), (L,))
    return k(idx_users.astype(jnp.int32), idx_items.astype(jnp.int32),
             user_emb_mf.T, item_emb_mf.T, w16, b16)
